# baseline (speedup 1.0000x reference)
import functools

import jax
import jax.numpy as jnp
from jax import lax
from jax.experimental import pallas as pl
from jax.experimental.pallas import tpu as pltpu
from jax.experimental.pallas import tpu_sc as plsc

_T = 10
_N = 10000
_E = 320000
_LEN_P1 = 5
_INNER = _T * (_T - 1)
_CROSS = _T * _N
_EDGES = _INNER + _E + 2 * _CROSS

_S1 = _INNER
_S2 = _S1 + _E
_S3 = _S2 + _CROSS

_NWORKERS = 32
_WV_PAD = 10016
_PIECE = 640
_W_PER_W = 312
_REP_PER_W = 3136

_ANY = pl.ANY


def _tc_body(tokf_hbm, gxf_hbm, tpe_ref, gpe_ref, gei_ref, xf_hbm, pe_ref,
             ei_ref, sems):
    cps = [
        pltpu.make_async_copy(tokf_hbm, xf_hbm.at[pl.ds(0, _T * 128)],
                              sems.at[0]),
        pltpu.make_async_copy(gxf_hbm, xf_hbm.at[pl.ds(_T * 128, _N * 128)],
                              sems.at[1]),
    ]
    for cp in cps:
        cp.start()
    pe_ref[0:_T, :] = tpe_ref[...]
    pe_ref[_T:, :] = gpe_ref[...]
    k = lax.broadcasted_iota(jnp.int32, (1, _INNER), 1)
    i = k // (_T - 1)
    r = k % (_T - 1)
    j = r + (r >= i).astype(jnp.int32)
    ei_ref[0:1, 0:_S1] = i
    ei_ref[1:2, 0:_S1] = j
    ei_ref[:, _S1:_S2] = gei_ref[...] + _T
    c = lax.broadcasted_iota(jnp.int32, (1, _CROSS), 1)
    ei_ref[0:1, _S2:_S3] = c // _N
    ei_ref[1:2, _S2:_S3] = c % _N + _T
    ei_ref[0:1, _S3:_EDGES] = c // _T + _T
    ei_ref[1:2, _S3:_EDGES] = c % _T
    for cp in cps:
        cp.wait()


def _ew_tc_body(w_ref, rep_ref, sh_ref, ew_ref):
    idx = lax.broadcasted_iota(jnp.int32, (128,), 0)
    sh = jnp.broadcast_to(sh_ref[...], (128,))
    ew_ref[pl.ds(0, 128)] = jnp.where(idx < _S1, sh, 1.0)
    ew_ref[pl.ds(128, _S2 - 128)] = jnp.ones((_S2 - 128,), jnp.float32)
    for t in range(_T):
        ew_ref[pl.ds(_S2 + t * _N, _N)] = w_ref[...]
    ew_ref[pl.ds(_S3, _CROSS)] = rep_ref[...]


def _ew_sc_body(spd_hbm, par_hbm, w_hbm, rep_hbm,
                spd_v, param_v, piece_v, rep_v, wv_v, wv_sh):
    s = lax.axis_index("s")
    c = lax.axis_index("c")
    wid = s * 2 + c
    lane = lax.iota(jnp.int32, 16)

    pltpu.sync_copy(spd_hbm, spd_v)
    pltpu.sync_copy(par_hbm, param_v)

    base_k = s * _PIECE

    def piece_step(it, carry):
        k = base_k + it * 16 + lane
        idx = jnp.minimum(k, _N - 1)
        sp = plsc.load_gather(spd_v, [idx])
        sp = jnp.minimum(jnp.maximum(sp, 0), _LEN_P1 - 1)
        piece_v[pl.ds(it * 16, 16)] = plsc.load_gather(param_v, [sp])
        return carry

    @pl.when(s < 15)
    def _():
        lax.fori_loop(0, _PIECE // 16, piece_step, 0, unroll=8)
        pltpu.sync_copy(piece_v, wv_sh.at[pl.ds(base_k, _PIECE)])

    @pl.when(s == 15)
    def _():
        lax.fori_loop(0, (_WV_PAD - 15 * _PIECE) // 16, piece_step, 0,
                      unroll=8)
        pltpu.sync_copy(piece_v.at[pl.ds(0, _WV_PAD - 15 * _PIECE)],
                        wv_sh.at[pl.ds(base_k, _WV_PAD - 15 * _PIECE)])

    plsc.subcore_barrier()
    pltpu.sync_copy(wv_sh, wv_v)

    w_lo = wid * _W_PER_W

    @pl.when(wid < _NWORKERS - 1)
    def _():
        pltpu.sync_copy(wv_v.at[pl.ds(w_lo, _W_PER_W)],
                        w_hbm.at[pl.ds(w_lo, _W_PER_W)])

    @pl.when(wid == _NWORKERS - 1)
    def _():
        lo = (_NWORKERS - 1) * _W_PER_W
        pltpu.sync_copy(wv_v.at[pl.ds(lo, _N - lo)],
                        w_hbm.at[pl.ds(lo, _N - lo)])

    rep_lo = wid * _REP_PER_W

    def rep_step(it, carry):
        j = rep_lo + it * 16 + lane
        i4 = jnp.minimum(lax.div(j, _T), _N - 1)
        rep_v[pl.ds(it * 16, 16)] = plsc.load_gather(wv_v, [i4])
        return carry

    @pl.when(wid < _NWORKERS - 1)
    def _():
        lax.fori_loop(0, _REP_PER_W // 16, rep_step, 0, unroll=8)
        pltpu.sync_copy(rep_v.at[pl.ds(0, _REP_PER_W)],
                        rep_hbm.at[pl.ds(rep_lo, _REP_PER_W)])

    @pl.when(wid == _NWORKERS - 1)
    def _():
        tail = _CROSS - (_NWORKERS - 1) * _REP_PER_W
        lax.fori_loop(0, tail // 16, rep_step, 0, unroll=8)
        pltpu.sync_copy(rep_v.at[pl.ds(0, tail)],
                        rep_hbm.at[pl.ds((_NWORKERS - 1) * _REP_PER_W, tail)])


@functools.cache
def _ew_sc_call():
    return functools.partial(
        pl.kernel,
        mesh=plsc.VectorSubcoreMesh(core_axis_name="c", subcore_axis_name="s"),
        compiler_params=pltpu.CompilerParams(needs_layout_passes=False),
        out_type=(
            jax.ShapeDtypeStruct((_N,), jnp.float32),
            jax.ShapeDtypeStruct((_CROSS,), jnp.float32),
        ),
        scratch_types=[
            pltpu.VMEM((_N,), jnp.int32),
            pltpu.VMEM((16,), jnp.float32),
            pltpu.VMEM((_PIECE,), jnp.float32),
            pltpu.VMEM((_REP_PER_W,), jnp.float32),
            pltpu.VMEM((_WV_PAD,), jnp.float32),
            pltpu.VMEM_SHARED((_WV_PAD,), jnp.float32),
        ],
    )(_ew_sc_body)


def kernel(g_x, g_pe, g_edge_index, g_spd, g_root_n_index, token_list,
           token_pe, shared_edge_weight, edge_weight_param, de, drop_e):
    xf, pe, ei = pl.pallas_call(
        _tc_body,
        in_specs=[
            pl.BlockSpec(memory_space=_ANY),
            pl.BlockSpec(memory_space=_ANY),
            pl.BlockSpec(memory_space=pltpu.MemorySpace.VMEM),
            pl.BlockSpec(memory_space=pltpu.MemorySpace.VMEM),
            pl.BlockSpec(memory_space=pltpu.MemorySpace.VMEM),
        ],
        out_specs=(
            pl.BlockSpec(memory_space=_ANY),
            pl.BlockSpec(memory_space=pltpu.MemorySpace.VMEM),
            pl.BlockSpec(memory_space=pltpu.MemorySpace.VMEM),
        ),
        out_shape=(
            jax.ShapeDtypeStruct(((_T + _N) * 128,), jnp.float32),
            jax.ShapeDtypeStruct((_T + _N, 32), jnp.float32),
            jax.ShapeDtypeStruct((2, _EDGES), jnp.int32),
        ),
        scratch_shapes=[pltpu.SemaphoreType.DMA((2,))],
    )(jnp.reshape(token_list, (-1,)), jnp.reshape(g_x, (-1,)), token_pe,
      g_pe, g_edge_index)
    x = jnp.reshape(xf, (_T + _N, 128))

    param16 = jnp.concatenate([
        edge_weight_param.astype(jnp.float32),
        jnp.zeros((16 - _LEN_P1,), jnp.float32),
    ])
    w, rep = _ew_sc_call()(g_spd, param16)

    ew = pl.pallas_call(
        _ew_tc_body,
        out_shape=jax.ShapeDtypeStruct((_EDGES,), jnp.float32),
    )(w, rep, jnp.reshape(shared_edge_weight.astype(jnp.float32), (1,)))

    root = g_root_n_index + _T
    return (x, pe, ei, ew, root)


# speedup vs baseline: 4.1479x; 4.1479x over previous
import functools

import jax
import jax.numpy as jnp
from jax import lax
from jax.experimental import pallas as pl
from jax.experimental.pallas import tpu as pltpu
from jax.experimental.pallas import tpu_sc as plsc

_T = 10
_N = 10000
_E = 320000
_LEN_P1 = 5
_INNER = _T * (_T - 1)
_CROSS = _T * _N
_EDGES = _INNER + _E + 2 * _CROSS

_S1 = _INNER
_S2 = _S1 + _E
_S3 = _S2 + _CROSS

_NWORKERS = 32
_WV_PAD = 10016
_PIECE = 640
_W_PER_W = 312
_REP_PER_W = 3136

_ANY = pl.ANY


def _tc_body(tok_ref, gx_ref, tpe_ref, gpe_ref, gei_ref, x_ref, pe_ref,
             ei_ref):
    x_ref[0:_T, :] = tok_ref[...]
    x_ref[_T:, :] = gx_ref[...]
    pe_ref[0:_T, :] = tpe_ref[...]
    pe_ref[_T:, :] = gpe_ref[...]
    k = lax.broadcasted_iota(jnp.int32, (1, _INNER), 1)
    i = k // (_T - 1)
    r = k % (_T - 1)
    j = r + (r >= i).astype(jnp.int32)
    ei_ref[0:1, 0:_S1] = i
    ei_ref[1:2, 0:_S1] = j
    ei_ref[:, _S1:_S2] = gei_ref[...] + _T
    c = lax.broadcasted_iota(jnp.int32, (1, _CROSS), 1)
    ei_ref[0:1, _S2:_S3] = c // _N
    ei_ref[1:2, _S2:_S3] = c % _N + _T
    ei_ref[0:1, _S3:_EDGES] = c // _T + _T
    ei_ref[1:2, _S3:_EDGES] = c % _T


def _ew_tc_body(w_ref, rep_ref, sh_ref, ew_ref):
    idx = lax.broadcasted_iota(jnp.int32, (128,), 0)
    sh = jnp.broadcast_to(sh_ref[...], (128,))
    ew_ref[pl.ds(0, 128)] = jnp.where(idx < _S1, sh, 1.0)
    ew_ref[pl.ds(128, _S2 - 128)] = jnp.ones((_S2 - 128,), jnp.float32)
    for t in range(_T):
        ew_ref[pl.ds(_S2 + t * _N, _N)] = w_ref[...]
    ew_ref[pl.ds(_S3, _CROSS)] = rep_ref[...]


def _ew_sc_body(spd_hbm, par_hbm, w_hbm, rep_hbm,
                spd_v, param_v, piece_v, rep_v, wv_v, wv_sh):
    s = lax.axis_index("s")
    c = lax.axis_index("c")
    wid = s * 2 + c
    lane = lax.iota(jnp.int32, 16)

    pltpu.sync_copy(spd_hbm, spd_v)
    pltpu.sync_copy(par_hbm, param_v)

    base_k = s * _PIECE

    def piece_step(it, carry):
        k = base_k + it * 16 + lane
        idx = jnp.minimum(k, _N - 1)
        sp = plsc.load_gather(spd_v, [idx])
        sp = jnp.minimum(jnp.maximum(sp, 0), _LEN_P1 - 1)
        piece_v[pl.ds(it * 16, 16)] = plsc.load_gather(param_v, [sp])
        return carry

    @pl.when(s < 15)
    def _():
        lax.fori_loop(0, _PIECE // 16, piece_step, 0, unroll=8)
        pltpu.sync_copy(piece_v, wv_sh.at[pl.ds(base_k, _PIECE)])

    @pl.when(s == 15)
    def _():
        lax.fori_loop(0, (_WV_PAD - 15 * _PIECE) // 16, piece_step, 0,
                      unroll=8)
        pltpu.sync_copy(piece_v.at[pl.ds(0, _WV_PAD - 15 * _PIECE)],
                        wv_sh.at[pl.ds(base_k, _WV_PAD - 15 * _PIECE)])

    plsc.subcore_barrier()
    pltpu.sync_copy(wv_sh, wv_v)

    w_lo = wid * _W_PER_W

    @pl.when(wid < _NWORKERS - 1)
    def _():
        pltpu.sync_copy(wv_v.at[pl.ds(w_lo, _W_PER_W)],
                        w_hbm.at[pl.ds(w_lo, _W_PER_W)])

    @pl.when(wid == _NWORKERS - 1)
    def _():
        lo = (_NWORKERS - 1) * _W_PER_W
        pltpu.sync_copy(wv_v.at[pl.ds(lo, _N - lo)],
                        w_hbm.at[pl.ds(lo, _N - lo)])

    rep_lo = wid * _REP_PER_W

    def rep_step(it, carry):
        j = rep_lo + it * 16 + lane
        i4 = jnp.minimum(lax.div(j, _T), _N - 1)
        rep_v[pl.ds(it * 16, 16)] = plsc.load_gather(wv_v, [i4])
        return carry

    @pl.when(wid < _NWORKERS - 1)
    def _():
        lax.fori_loop(0, _REP_PER_W // 16, rep_step, 0, unroll=8)
        pltpu.sync_copy(rep_v.at[pl.ds(0, _REP_PER_W)],
                        rep_hbm.at[pl.ds(rep_lo, _REP_PER_W)])

    @pl.when(wid == _NWORKERS - 1)
    def _():
        tail = _CROSS - (_NWORKERS - 1) * _REP_PER_W
        lax.fori_loop(0, tail // 16, rep_step, 0, unroll=8)
        pltpu.sync_copy(rep_v.at[pl.ds(0, tail)],
                        rep_hbm.at[pl.ds((_NWORKERS - 1) * _REP_PER_W, tail)])


@functools.cache
def _ew_sc_call():
    return functools.partial(
        pl.kernel,
        mesh=plsc.VectorSubcoreMesh(core_axis_name="c", subcore_axis_name="s"),
        compiler_params=pltpu.CompilerParams(needs_layout_passes=False),
        out_type=(
            jax.ShapeDtypeStruct((_N,), jnp.float32),
            jax.ShapeDtypeStruct((_CROSS,), jnp.float32),
        ),
        scratch_types=[
            pltpu.VMEM((_N,), jnp.int32),
            pltpu.VMEM((16,), jnp.float32),
            pltpu.VMEM((_PIECE,), jnp.float32),
            pltpu.VMEM((_REP_PER_W,), jnp.float32),
            pltpu.VMEM((_WV_PAD,), jnp.float32),
            pltpu.VMEM_SHARED((_WV_PAD,), jnp.float32),
        ],
    )(_ew_sc_body)


def kernel(g_x, g_pe, g_edge_index, g_spd, g_root_n_index, token_list,
           token_pe, shared_edge_weight, edge_weight_param, de, drop_e):
    x, pe, ei = pl.pallas_call(
        _tc_body,
        out_shape=(
            jax.ShapeDtypeStruct((_T + _N, 128), jnp.float32),
            jax.ShapeDtypeStruct((_T + _N, 32), jnp.float32),
            jax.ShapeDtypeStruct((2, _EDGES), jnp.int32),
        ),
    )(token_list, g_x, token_pe, g_pe, g_edge_index)

    param16 = jnp.concatenate([
        edge_weight_param.astype(jnp.float32),
        jnp.zeros((16 - _LEN_P1,), jnp.float32),
    ])
    w, rep = _ew_sc_call()(g_spd, param16)

    ew = pl.pallas_call(
        _ew_tc_body,
        out_shape=jax.ShapeDtypeStruct((_EDGES,), jnp.float32),
    )(w, rep, jnp.reshape(shared_edge_weight.astype(jnp.float32), (1,)))

    root = g_root_n_index + _T
    return (x, pe, ei, ew, root)
